# baseline (speedup 1.0000x reference)
import functools

import jax
import jax.numpy as jnp
from jax import lax
from jax.experimental import pallas as pl
from jax.experimental.pallas import tpu as pltpu
from jax.experimental.pallas import tpu_sc as plsc

_GRID = 1024
_BAND_ROWS = 32
_PADN = 204800
_ROWS = _PADN // 128

_NC = 2
_NS = 16
_NW = _NC * _NS


def _bin_keys(px2, py2, n_valid):
    def body(px_ref, py_ref, key_ref):
        x = px_ref[...]
        y = py_ref[...]
        cx = ((x - jnp.float32(-51.2)) / jnp.float32(0.1)).astype(jnp.int32)
        cy = ((y - jnp.float32(-51.2)) / jnp.float32(0.1)).astype(jnp.int32)
        key = (cy << 10) | cx
        idx = (lax.broadcasted_iota(jnp.int32, (_ROWS, 128), 0) * 128
               + lax.broadcasted_iota(jnp.int32, (_ROWS, 128), 1))
        key_ref[...] = jnp.where(idx < n_valid, key, -1)

    return pl.pallas_call(
        body,
        out_shape=jax.ShapeDtypeStruct((_ROWS, 128), jnp.int32),
    )(px2, py2)


_TILEK = _PADN // _NS
_SCHUNK = 1280
_NSCH = _TILEK // _SCHUNK
_HALF = 512 * _GRID
_SHW = _HALF + 16


@functools.cache
def _make_count_kernel():
    mesh = plsc.VectorSubcoreMesh(
        core_axis_name="c", subcore_axis_name="s",
        num_cores=_NC, num_subcores=_NS)

    @functools.partial(
        pl.kernel,
        mesh=mesh,
        out_type=jax.ShapeDtypeStruct((_NW, 16), jnp.float32),
        compiler_params=pltpu.CompilerParams(needs_layout_passes=False),
        scratch_types=[
            pltpu.VMEM_SHARED((_SHW,), jnp.float32),
            pltpu.VMEM((2, _SCHUNK), jnp.int32),
            pltpu.VMEM((_SCHUNK,), jnp.int32),
            pltpu.VMEM((_SCHUNK,), jnp.int32),
            pltpu.VMEM((_SCHUNK,), jnp.float32),
            pltpu.VMEM((4096,), jnp.float32),
            pltpu.VMEM((_BAND_ROWS * _GRID,), jnp.float32),
            pltpu.VMEM((16 * 512,), jnp.float32),
            pltpu.VMEM((8 * 256,), jnp.float32),
            pltpu.VMEM((4 * 128,), jnp.float32),
            pltpu.VMEM((2 * 64,), jnp.float32),
            pltpu.VMEM((1 * 32,), jnp.float32),
            pltpu.VMEM((16,), jnp.float32),
            pltpu.SemaphoreType.DMA,
            pltpu.SemaphoreType.DMA,
            pltpu.SemaphoreType.DMA,
            pltpu.SemaphoreType.DMA,
        ],
    )
    def count_kernel(keys_hbm, out_hbm, grid, kbuf, idxb0, idxb1, onesb,
                     zb, occ, p1, p2, p3, p4, p5, res,
                     sem0, sem1, ssem0, ssem1):
        idxbs = (idxb0, idxb1)
        cid = lax.axis_index("c")
        sid = lax.axis_index("s")
        wid = cid * _NS + sid
        lanes = lax.iota(jnp.int32, 16)
        zero16 = jnp.zeros((16,), jnp.float32)
        ones16 = jnp.ones((16,), jnp.float32)
        hbase = cid * _HALF

        sems = (sem0, sem1)
        ssems = (ssem0, ssem1)

        pltpu.async_copy(keys_hbm.at[sid, 0], kbuf.at[0], sem0)

        @plsc.parallel_loop(0, _SCHUNK // 16, unroll=8)
        def _ones(g):
            onesb[pl.ds(g * 16, 16)] = ones16

        @plsc.parallel_loop(0, 4096 // 16, unroll=8)
        def _zb(g):
            zb[pl.ds(g * 16, 16)] = zero16

        for j in range(8):
            pltpu.sync_copy(zb, grid.at[pl.ds(sid * 32768 + j * 4096,
                                              4096)])

        @pl.when(sid == 0)
        def _():
            pltpu.sync_copy(zb.at[pl.ds(0, 16)], grid.at[pl.ds(_HALF, 16)])

        plsc.subcore_barrier()

        for ch in range(_NSCH):
            b = ch & 1
            pltpu.make_async_copy(keys_hbm.at[sid, ch], kbuf.at[b],
                                  sems[b]).wait()
            if ch + 1 < _NSCH:
                pltpu.async_copy(keys_hbm.at[sid, ch + 1], kbuf.at[1 - b],
                                 sems[1 - b])
            if ch >= 2:
                pltpu.make_async_copy(
                    onesb, grid.at[idxbs[b]], ssems[b]).wait()

            @plsc.parallel_loop(0, _SCHUNK // 16, unroll=8)
            def _mkidx(g):
                k = kbuf[b, pl.ds(g * 16, 16)]
                a = k - hbase
                au = plsc.bitcast(a, jnp.uint32)
                idx = plsc.bitcast(
                    jnp.minimum(au, jnp.uint32(_HALF)), jnp.int32)
                idxbs[b][pl.ds(g * 16, 16)] = idx

            pltpu.async_copy(onesb, grid.at[idxbs[b]], ssems[b],
                             add=True)

        pltpu.make_async_copy(onesb, grid.at[idxb0], ssems[0]).wait()
        pltpu.make_async_copy(onesb, grid.at[idxb1], ssems[1]).wait()

        plsc.subcore_barrier()

        pltpu.sync_copy(grid.at[pl.ds(sid * 32768, 32768)], occ)

        def pool(src, dst, hd, wd, counts):
            gpr = wd // 16
            lg = gpr.bit_length() - 1
            s = 2 * wd
            iota2 = lanes * 2

            @plsc.parallel_loop(0, hd * gpr, unroll=4,
                                carry=(zero16, zero16))
            def body(cc, carry):
                accm, accs = carry
                yy = lax.shift_right_logical(cc, lg)
                j = lax.bitwise_and(cc, gpr - 1)
                base = yy * (2 * s) + j * 32 + iota2
                a = plsc.load_gather(src, [base])
                b2 = plsc.load_gather(src, [base + 1])
                e = plsc.load_gather(src, [base + s])
                f = plsc.load_gather(src, [base + s + 1])
                if counts:
                    a = jnp.where(a > 0.0, 1.0, 0.0)
                    b2 = jnp.where(b2 > 0.0, 1.0, 0.0)
                    e = jnp.where(e > 0.0, 1.0, 0.0)
                    f = jnp.where(f > 0.0, 1.0, 0.0)
                m = jnp.maximum(jnp.maximum(a, b2), jnp.maximum(e, f))
                dst[pl.ds(cc * 16, 16)] = m
                accm = accm + m
                if counts:
                    accs = accs + ((a + b2) + (e + f))
                return (accm, accs)

            return body

        acc1, acc0 = pool(occ, p1, 16, 512, True)
        acc2, _ = pool(p1, p2, 8, 256, False)
        acc3, _ = pool(p2, p3, 4, 128, False)
        acc4, _ = pool(p3, p4, 2, 64, False)
        acc5, _ = pool(p4, p5, 1, 32, False)

        resv = zero16
        for k_idx, acc in enumerate((acc0, acc1, acc2, acc3, acc4, acc5)):
            t = jnp.sum(acc)
            resv = jnp.where(lanes == k_idx, jnp.broadcast_to(t, (16,)),
                             resv)
        res[...] = resv
        pltpu.sync_copy(res, out_hbm.at[wid])

    return count_kernel


def _combine(t, tt):
    def body(t_ref, tt_ref, pc0_ref, cnt_ref):
        tm = t_ref[...]
        tmt = tt_ref[...]

        tot = jnp.sum(tm, axis=0, keepdims=True)
        pc0_ref[...] = tot[:, 0:3]

        c0 = tmt[0:1] + tmt[1:2] + tmt[2:3] + tmt[3:4]
        av = tm[:, 1:2] + tm[:, 2:3] + tm[:, 3:4] + tm[:, 4:5]
        bv = tm[:, 2:3] + tm[:, 3:4] + tm[:, 4:5] + tm[:, 5:6]
        jj = lax.broadcasted_iota(jnp.int32, (32, 32), 0)
        ss = lax.broadcasted_iota(jnp.int32, (32, 32), 1)
        m1 = ((jj >> 1) == ss).astype(jnp.float32)
        m2 = ((jj >> 2) == ss).astype(jnp.float32)
        c1 = jnp.sum(av * m1, axis=0, keepdims=True)
        c2 = jnp.sum(bv * m2, axis=0, keepdims=True)
        cnt_ref[...] = jnp.concatenate([c0, c1, c2], axis=0)

    return pl.pallas_call(
        body,
        out_shape=[
            jax.ShapeDtypeStruct((1, 3), jnp.float32),
            jax.ShapeDtypeStruct((3, 32), jnp.float32),
        ],
    )(t, tt)


def kernel(points_inds, first_res_idx):
    del first_res_idx
    pts = points_inds
    n = pts.shape[0]
    px = jnp.pad(pts[:, 0], (0, _PADN - n))
    py = jnp.pad(pts[:, 1], (0, _PADN - n))
    keys = _bin_keys(px.reshape(_ROWS, 128), py.reshape(_ROWS, 128), n)
    t = _make_count_kernel()(keys.reshape(_NS, _NSCH, _SCHUNK))
    pc0, counts = _combine(t, t.T)
    return pc0, counts


# speedup vs baseline: 1.4680x; 1.4680x over previous
import functools

import jax
import jax.numpy as jnp
from jax import lax
from jax.experimental import pallas as pl
from jax.experimental.pallas import tpu as pltpu
from jax.experimental.pallas import tpu_sc as plsc

_GRID = 1024
_BAND_ROWS = 32
_PADN = 204800
_ROWS = _PADN // 128
_CHUNK = 2048
_NCHUNK = _PADN // _CHUNK

_NC = 2
_NS = 16
_NW = _NC * _NS


def _bin_keys(px2, py2, n_valid):
    def body(px_ref, py_ref, key_ref):
        x = px_ref[...]
        y = py_ref[...]
        cx = ((x - jnp.float32(-51.2)) / jnp.float32(0.1)).astype(jnp.int32)
        cy = ((y - jnp.float32(-51.2)) / jnp.float32(0.1)).astype(jnp.int32)
        key = (cy << 10) | cx
        idx = (lax.broadcasted_iota(jnp.int32, (_ROWS, 128), 0) * 128
               + lax.broadcasted_iota(jnp.int32, (_ROWS, 128), 1))
        key_ref[...] = jnp.where(idx < n_valid, key, -1)

    return pl.pallas_call(
        body,
        out_shape=jax.ShapeDtypeStruct((_ROWS, 128), jnp.int32),
    )(px2, py2)


@functools.cache
def _make_count_kernel():
    mesh = plsc.VectorSubcoreMesh(
        core_axis_name="c", subcore_axis_name="s",
        num_cores=_NC, num_subcores=_NS)

    @functools.partial(
        pl.kernel,
        mesh=mesh,
        out_type=jax.ShapeDtypeStruct((_NW, 16), jnp.float32),
        compiler_params=pltpu.CompilerParams(needs_layout_passes=False),
        scratch_types=[
            pltpu.VMEM((2, _CHUNK), jnp.int32),
            pltpu.VMEM((_BAND_ROWS * _GRID + 16,), jnp.float32),
            pltpu.VMEM((16 * 512,), jnp.float32),
            pltpu.VMEM((8 * 256,), jnp.float32),
            pltpu.VMEM((4 * 128,), jnp.float32),
            pltpu.VMEM((2 * 64,), jnp.float32),
            pltpu.VMEM((1 * 32,), jnp.float32),
            pltpu.VMEM((16,), jnp.float32),
            pltpu.SemaphoreType.DMA,
            pltpu.SemaphoreType.DMA,
        ],
    )
    def count_kernel(keys_hbm, out_hbm, kbuf, occ, p1, p2, p3, p4, p5,
                     res, sem0, sem1):
        wid = lax.axis_index("s") * _NC + lax.axis_index("c")
        wbase = wid * (_BAND_ROWS * _GRID)
        lanes = lax.iota(jnp.int32, 16)
        zero16 = jnp.zeros((16,), jnp.float32)
        ones16 = jnp.ones((16,), jnp.float32)

        @plsc.parallel_loop(0, (_BAND_ROWS * _GRID) // 16, unroll=8)
        def _zero(i):
            occ[pl.ds(i * 16, 16)] = zero16

        sems = (sem0, sem1)
        pltpu.async_copy(keys_hbm.at[0], kbuf.at[0], sem0)

        def chunk_body(h, _):
            for b in range(2):
                c = h * 2 + b
                pltpu.make_async_copy(keys_hbm.at[c], kbuf.at[b],
                                      sems[b]).wait()

                @pl.when(c + 1 < _NCHUNK)
                def _():
                    pltpu.async_copy(keys_hbm.at[c + 1], kbuf.at[1 - b],
                                     sems[1 - b])

                @plsc.parallel_loop(0, _CHUNK // 16, unroll=8)
                def _scan(j):
                    k = kbuf[b, pl.ds(j * 16, 16)]
                    a = k - wbase
                    au = plsc.bitcast(a, jnp.uint32)
                    addr = plsc.bitcast(
                        jnp.minimum(au, jnp.uint32(_BAND_ROWS * _GRID)),
                        jnp.int32)
                    plsc.store_scatter(occ, [addr], ones16)
            return 0
        lax.fori_loop(0, _NCHUNK // 2, chunk_body, 0)

        def pool(src, dst, hd, wd, with_sum):
            gpr = wd // 16
            lg = gpr.bit_length() - 1
            s = 2 * wd
            iota2 = lanes * 2

            @plsc.parallel_loop(0, hd * gpr, unroll=4,
                                carry=(zero16, zero16))
            def body(cc, carry):
                accm, accs = carry
                yy = lax.shift_right_logical(cc, lg)
                j = lax.bitwise_and(cc, gpr - 1)
                base = yy * (2 * s) + j * 32 + iota2
                a = plsc.load_gather(src, [base])
                b2 = plsc.load_gather(src, [base + 1])
                e = plsc.load_gather(src, [base + s])
                f = plsc.load_gather(src, [base + s + 1])
                m = jnp.maximum(jnp.maximum(a, b2), jnp.maximum(e, f))
                dst[pl.ds(cc * 16, 16)] = m
                accm = accm + m
                if with_sum:
                    accs = accs + ((a + b2) + (e + f))
                return (accm, accs)

            return body

        acc1, acc0 = pool(occ, p1, 16, 512, True)
        acc2, _ = pool(p1, p2, 8, 256, False)
        acc3, _ = pool(p2, p3, 4, 128, False)
        acc4, _ = pool(p3, p4, 2, 64, False)
        acc5, _ = pool(p4, p5, 1, 32, False)

        resv = zero16
        for k_idx, acc in enumerate((acc0, acc1, acc2, acc3, acc4, acc5)):
            t = jnp.sum(acc)
            resv = jnp.where(lanes == k_idx, jnp.broadcast_to(t, (16,)),
                             resv)
        res[...] = resv
        pltpu.sync_copy(res, out_hbm.at[wid])

    return count_kernel


def _combine(t, tt):
    def body(t_ref, tt_ref, pc0_ref, cnt_ref):
        tm = t_ref[...]
        tmt = tt_ref[...]

        tot = jnp.sum(tm, axis=0, keepdims=True)
        pc0_ref[...] = tot[:, 0:3]

        c0 = tmt[0:1] + tmt[1:2] + tmt[2:3] + tmt[3:4]
        av = tm[:, 1:2] + tm[:, 2:3] + tm[:, 3:4] + tm[:, 4:5]
        bv = tm[:, 2:3] + tm[:, 3:4] + tm[:, 4:5] + tm[:, 5:6]
        jj = lax.broadcasted_iota(jnp.int32, (32, 32), 0)
        ss = lax.broadcasted_iota(jnp.int32, (32, 32), 1)
        m1 = ((jj >> 1) == ss).astype(jnp.float32)
        m2 = ((jj >> 2) == ss).astype(jnp.float32)
        c1 = jnp.sum(av * m1, axis=0, keepdims=True)
        c2 = jnp.sum(bv * m2, axis=0, keepdims=True)
        cnt_ref[...] = jnp.concatenate([c0, c1, c2], axis=0)

    return pl.pallas_call(
        body,
        out_shape=[
            jax.ShapeDtypeStruct((1, 3), jnp.float32),
            jax.ShapeDtypeStruct((3, 32), jnp.float32),
        ],
    )(t, tt)


def kernel(points_inds, first_res_idx):
    del first_res_idx
    pts = points_inds
    n = pts.shape[0]
    px = jnp.pad(pts[:, 0], (0, _PADN - n))
    py = jnp.pad(pts[:, 1], (0, _PADN - n))
    keys = _bin_keys(px.reshape(_ROWS, 128), py.reshape(_ROWS, 128), n)
    t = _make_count_kernel()(keys.reshape(_NCHUNK, _CHUNK))
    pc0, counts = _combine(t, t.T)
    return pc0, counts


# speedup vs baseline: 1.9050x; 1.2977x over previous
import functools

import jax
import jax.numpy as jnp
from jax import lax
from jax.experimental import pallas as pl
from jax.experimental.pallas import tpu as pltpu
from jax.experimental.pallas import tpu_sc as plsc

_GRID = 1024
_BAND_ROWS = 32
_PADN = 204800
_ROWS = _PADN // 128
_CHUNK = 4096
_NCHUNK = _PADN // _CHUNK

_NC = 2
_NS = 16
_NW = _NC * _NS


def _bin_keys(px2, py2, n_valid):
    def body(px_ref, py_ref, key_ref):
        x = px_ref[...]
        y = py_ref[...]
        cx = ((x - jnp.float32(-51.2)) / jnp.float32(0.1)).astype(jnp.int32)
        cy = ((y - jnp.float32(-51.2)) / jnp.float32(0.1)).astype(jnp.int32)
        key = (cy << 10) | cx
        idx = (lax.broadcasted_iota(jnp.int32, (_ROWS, 128), 0) * 128
               + lax.broadcasted_iota(jnp.int32, (_ROWS, 128), 1))
        key_ref[...] = jnp.where(idx < n_valid, key, -1)

    return pl.pallas_call(
        body,
        out_shape=jax.ShapeDtypeStruct((_ROWS, 128), jnp.int32),
    )(px2, py2)


@functools.cache
def _make_count_kernel():
    mesh = plsc.VectorSubcoreMesh(
        core_axis_name="c", subcore_axis_name="s",
        num_cores=_NC, num_subcores=_NS)

    @functools.partial(
        pl.kernel,
        mesh=mesh,
        out_type=jax.ShapeDtypeStruct((_NW, 16), jnp.float32),
        compiler_params=pltpu.CompilerParams(needs_layout_passes=False),
        scratch_types=[
            pltpu.VMEM((2, _CHUNK), jnp.int32),
            pltpu.VMEM((_BAND_ROWS * _GRID + 16,), jnp.float32),
            pltpu.VMEM((16 * 512,), jnp.float32),
            pltpu.VMEM((8 * 256,), jnp.float32),
            pltpu.VMEM((4 * 128,), jnp.float32),
            pltpu.VMEM((2 * 64,), jnp.float32),
            pltpu.VMEM((1 * 32,), jnp.float32),
            pltpu.VMEM((16,), jnp.float32),
            pltpu.SemaphoreType.DMA,
            pltpu.SemaphoreType.DMA,
        ],
    )
    def count_kernel(keys_hbm, out_hbm, kbuf, occ, p1, p2, p3, p4, p5,
                     res, sem0, sem1):
        wid = lax.axis_index("s") * _NC + lax.axis_index("c")
        wbase = wid * (_BAND_ROWS * _GRID)
        lanes = lax.iota(jnp.int32, 16)
        zero16 = jnp.zeros((16,), jnp.float32)
        ones16 = jnp.ones((16,), jnp.float32)

        @plsc.parallel_loop(0, (_BAND_ROWS * _GRID) // 16, unroll=8)
        def _zero(i):
            occ[pl.ds(i * 16, 16)] = zero16

        sems = (sem0, sem1)
        pltpu.async_copy(keys_hbm.at[0], kbuf.at[0], sem0)

        def chunk_body(h, _):
            for b in range(2):
                c = h * 2 + b
                pltpu.make_async_copy(keys_hbm.at[c], kbuf.at[b],
                                      sems[b]).wait()

                @pl.when(c + 1 < _NCHUNK)
                def _():
                    pltpu.async_copy(keys_hbm.at[c + 1], kbuf.at[1 - b],
                                     sems[1 - b])

                @plsc.parallel_loop(0, _CHUNK // 16, unroll=16)
                def _scan(j):
                    k = kbuf[b, pl.ds(j * 16, 16)]
                    a = k - wbase
                    au = plsc.bitcast(a, jnp.uint32)
                    addr = plsc.bitcast(
                        jnp.minimum(au, jnp.uint32(_BAND_ROWS * _GRID)),
                        jnp.int32)
                    plsc.store_scatter(occ, [addr], ones16)
            return 0
        lax.fori_loop(0, _NCHUNK // 2, chunk_body, 0)

        def pool(src, dst, hd, wd, with_sum):
            gpr = wd // 16
            lg = gpr.bit_length() - 1
            s = 2 * wd
            iota2 = lanes * 2

            @plsc.parallel_loop(0, hd * gpr, unroll=4,
                                carry=(zero16, zero16))
            def body(cc, carry):
                accm, accs = carry
                yy = lax.shift_right_logical(cc, lg)
                j = lax.bitwise_and(cc, gpr - 1)
                base = yy * (2 * s) + j * 32 + iota2
                a = plsc.load_gather(src, [base])
                b2 = plsc.load_gather(src, [base + 1])
                e = plsc.load_gather(src, [base + s])
                f = plsc.load_gather(src, [base + s + 1])
                m = jnp.maximum(jnp.maximum(a, b2), jnp.maximum(e, f))
                dst[pl.ds(cc * 16, 16)] = m
                accm = accm + m
                if with_sum:
                    accs = accs + ((a + b2) + (e + f))
                return (accm, accs)

            return body

        acc1, acc0 = pool(occ, p1, 16, 512, True)
        acc2, _ = pool(p1, p2, 8, 256, False)
        acc3, _ = pool(p2, p3, 4, 128, False)
        acc4, _ = pool(p3, p4, 2, 64, False)
        acc5, _ = pool(p4, p5, 1, 32, False)

        resv = zero16
        for k_idx, acc in enumerate((acc0, acc1, acc2, acc3, acc4, acc5)):
            t = jnp.sum(acc)
            resv = jnp.where(lanes == k_idx, jnp.broadcast_to(t, (16,)),
                             resv)
        res[...] = resv
        pltpu.sync_copy(res, out_hbm.at[wid])

    return count_kernel


def _combine(t, tt):
    def body(t_ref, tt_ref, pc0_ref, cnt_ref):
        tm = t_ref[...]
        tmt = tt_ref[...]

        tot = jnp.sum(tm, axis=0, keepdims=True)
        pc0_ref[...] = tot[:, 0:3]

        c0 = tmt[0:1] + tmt[1:2] + tmt[2:3] + tmt[3:4]
        av = tm[:, 1:2] + tm[:, 2:3] + tm[:, 3:4] + tm[:, 4:5]
        bv = tm[:, 2:3] + tm[:, 3:4] + tm[:, 4:5] + tm[:, 5:6]
        jj = lax.broadcasted_iota(jnp.int32, (32, 32), 0)
        ss = lax.broadcasted_iota(jnp.int32, (32, 32), 1)
        m1 = ((jj >> 1) == ss).astype(jnp.float32)
        m2 = ((jj >> 2) == ss).astype(jnp.float32)
        c1 = jnp.sum(av * m1, axis=0, keepdims=True)
        c2 = jnp.sum(bv * m2, axis=0, keepdims=True)
        cnt_ref[...] = jnp.concatenate([c0, c1, c2], axis=0)

    return pl.pallas_call(
        body,
        out_shape=[
            jax.ShapeDtypeStruct((1, 3), jnp.float32),
            jax.ShapeDtypeStruct((3, 32), jnp.float32),
        ],
    )(t, tt)


def kernel(points_inds, first_res_idx):
    del first_res_idx
    pts = points_inds
    n = pts.shape[0]
    px = jnp.pad(pts[:, 0], (0, _PADN - n))
    py = jnp.pad(pts[:, 1], (0, _PADN - n))
    keys = _bin_keys(px.reshape(_ROWS, 128), py.reshape(_ROWS, 128), n)
    t = _make_count_kernel()(keys.reshape(_NCHUNK, _CHUNK))
    pc0, counts = _combine(t, t.T)
    return pc0, counts


# speedup vs baseline: 2.0834x; 1.0937x over previous
import functools

import jax
import jax.numpy as jnp
from jax import lax
from jax.experimental import pallas as pl
from jax.experimental.pallas import tpu as pltpu
from jax.experimental.pallas import tpu_sc as plsc

_GRID = 1024
_BAND_ROWS = 32
_PADN = 204800
_ROWS = _PADN // 128
_CHUNK = 6400
_NCHUNK = _PADN // _CHUNK

_NC = 2
_NS = 16
_NW = _NC * _NS


def _bin_keys(px2, py2, n_valid):
    def body(px_ref, py_ref, key_ref):
        x = px_ref[...]
        y = py_ref[...]
        cx = ((x - jnp.float32(-51.2)) / jnp.float32(0.1)).astype(jnp.int32)
        cy = ((y - jnp.float32(-51.2)) / jnp.float32(0.1)).astype(jnp.int32)
        key = (cy << 10) | cx
        idx = (lax.broadcasted_iota(jnp.int32, (_ROWS, 128), 0) * 128
               + lax.broadcasted_iota(jnp.int32, (_ROWS, 128), 1))
        key_ref[...] = jnp.where(idx < n_valid, key, -1)

    return pl.pallas_call(
        body,
        out_shape=jax.ShapeDtypeStruct((_ROWS, 128), jnp.int32),
    )(px2, py2)


@functools.cache
def _make_count_kernel():
    mesh = plsc.VectorSubcoreMesh(
        core_axis_name="c", subcore_axis_name="s",
        num_cores=_NC, num_subcores=_NS)

    @functools.partial(
        pl.kernel,
        mesh=mesh,
        out_type=jax.ShapeDtypeStruct((_NW, 16), jnp.float32),
        compiler_params=pltpu.CompilerParams(needs_layout_passes=False),
        scratch_types=[
            pltpu.VMEM((2, _CHUNK), jnp.int32),
            pltpu.VMEM((_BAND_ROWS * _GRID + 16,), jnp.float32),
            pltpu.VMEM((16 * 512,), jnp.float32),
            pltpu.VMEM((8 * 256,), jnp.float32),
            pltpu.VMEM((4 * 128,), jnp.float32),
            pltpu.VMEM((2 * 64,), jnp.float32),
            pltpu.VMEM((1 * 32,), jnp.float32),
            pltpu.VMEM((16,), jnp.float32),
            pltpu.SemaphoreType.DMA,
            pltpu.SemaphoreType.DMA,
        ],
    )
    def count_kernel(keys_hbm, out_hbm, kbuf, occ, p1, p2, p3, p4, p5,
                     res, sem0, sem1):
        wid = lax.axis_index("s") * _NC + lax.axis_index("c")
        wbase = wid * (_BAND_ROWS * _GRID)
        lanes = lax.iota(jnp.int32, 16)
        zero16 = jnp.zeros((16,), jnp.float32)
        ones16 = jnp.ones((16,), jnp.float32)

        @plsc.parallel_loop(0, (_BAND_ROWS * _GRID) // 16, unroll=16)
        def _zero(i):
            occ[pl.ds(i * 16, 16)] = zero16

        sems = (sem0, sem1)
        pltpu.async_copy(keys_hbm.at[0], kbuf.at[0], sem0)

        def chunk_body(h, _):
            for b in range(2):
                c = h * 2 + b
                pltpu.make_async_copy(keys_hbm.at[c], kbuf.at[b],
                                      sems[b]).wait()

                @pl.when(c + 1 < _NCHUNK)
                def _():
                    pltpu.async_copy(keys_hbm.at[c + 1], kbuf.at[1 - b],
                                     sems[1 - b])

                @plsc.parallel_loop(0, _CHUNK // 16, unroll=16)
                def _scan(j):
                    k = kbuf[b, pl.ds(j * 16, 16)]
                    a = k - wbase
                    au = plsc.bitcast(a, jnp.uint32)
                    addr = plsc.bitcast(
                        jnp.minimum(au, jnp.uint32(_BAND_ROWS * _GRID)),
                        jnp.int32)
                    plsc.store_scatter(occ, [addr], ones16)
            return 0
        lax.fori_loop(0, _NCHUNK // 2, chunk_body, 0)

        def pool(src, dst, hd, wd, with_sum):
            gpr = wd // 16
            lg = gpr.bit_length() - 1
            s = 2 * wd
            iota2 = lanes * 2

            @plsc.parallel_loop(0, hd * gpr, unroll=8 if gpr >= 8 else 2,
                                carry=(zero16, zero16))
            def body(cc, carry):
                accm, accs = carry
                yy = lax.shift_right_logical(cc, lg)
                j = lax.bitwise_and(cc, gpr - 1)
                base = yy * (2 * s) + j * 32 + iota2
                a = plsc.load_gather(src, [base])
                b2 = plsc.load_gather(src, [base + 1])
                e = plsc.load_gather(src, [base + s])
                f = plsc.load_gather(src, [base + s + 1])
                m = jnp.maximum(jnp.maximum(a, b2), jnp.maximum(e, f))
                dst[pl.ds(cc * 16, 16)] = m
                accm = accm + m
                if with_sum:
                    accs = accs + ((a + b2) + (e + f))
                return (accm, accs)

            return body

        acc1, acc0 = pool(occ, p1, 16, 512, True)
        acc2, _ = pool(p1, p2, 8, 256, False)
        acc3, _ = pool(p2, p3, 4, 128, False)
        acc4, _ = pool(p3, p4, 2, 64, False)
        acc5, _ = pool(p4, p5, 1, 32, False)

        resv = zero16
        for k_idx, acc in enumerate((acc0, acc1, acc2, acc3, acc4, acc5)):
            t = jnp.sum(acc)
            resv = jnp.where(lanes == k_idx, jnp.broadcast_to(t, (16,)),
                             resv)
        res[...] = resv
        pltpu.sync_copy(res, out_hbm.at[wid])

    return count_kernel


def _combine(t, tt):
    def body(t_ref, tt_ref, pc0_ref, cnt_ref):
        tm = t_ref[...]
        tmt = tt_ref[...]

        tot = jnp.sum(tm, axis=0, keepdims=True)
        pc0_ref[...] = tot[:, 0:3]

        c0 = tmt[0:1] + tmt[1:2] + tmt[2:3] + tmt[3:4]
        av = tm[:, 1:2] + tm[:, 2:3] + tm[:, 3:4] + tm[:, 4:5]
        bv = tm[:, 2:3] + tm[:, 3:4] + tm[:, 4:5] + tm[:, 5:6]
        jj = lax.broadcasted_iota(jnp.int32, (32, 32), 0)
        ss = lax.broadcasted_iota(jnp.int32, (32, 32), 1)
        m1 = ((jj >> 1) == ss).astype(jnp.float32)
        m2 = ((jj >> 2) == ss).astype(jnp.float32)
        c1 = jnp.sum(av * m1, axis=0, keepdims=True)
        c2 = jnp.sum(bv * m2, axis=0, keepdims=True)
        cnt_ref[...] = jnp.concatenate([c0, c1, c2], axis=0)

    return pl.pallas_call(
        body,
        out_shape=[
            jax.ShapeDtypeStruct((1, 3), jnp.float32),
            jax.ShapeDtypeStruct((3, 32), jnp.float32),
        ],
    )(t, tt)


def kernel(points_inds, first_res_idx):
    del first_res_idx
    pts = points_inds
    n = pts.shape[0]
    px = jnp.pad(pts[:, 0], (0, _PADN - n))
    py = jnp.pad(pts[:, 1], (0, _PADN - n))
    keys = _bin_keys(px.reshape(_ROWS, 128), py.reshape(_ROWS, 128), n)
    t = _make_count_kernel()(keys.reshape(_NCHUNK, _CHUNK))
    pc0, counts = _combine(t, t.T)
    return pc0, counts


# speedup vs baseline: 2.2039x; 1.0579x over previous
import functools

import jax
import jax.numpy as jnp
from jax import lax
from jax.experimental import pallas as pl
from jax.experimental.pallas import tpu as pltpu
from jax.experimental.pallas import tpu_sc as plsc

_GRID = 1024
_BAND_ROWS = 32
_PADN = 204800
_ROWS = _PADN // 128
_CHUNK = 12800
_NCHUNK = _PADN // _CHUNK

_NC = 2
_NS = 16
_NW = _NC * _NS


def _bin_keys(px2, py2, n_valid):
    def body(px_ref, py_ref, key_ref):
        x = px_ref[...]
        y = py_ref[...]
        cx = ((x - jnp.float32(-51.2)) / jnp.float32(0.1)).astype(jnp.int32)
        cy = ((y - jnp.float32(-51.2)) / jnp.float32(0.1)).astype(jnp.int32)
        key = (cy << 10) | cx
        idx = (lax.broadcasted_iota(jnp.int32, (_ROWS, 128), 0) * 128
               + lax.broadcasted_iota(jnp.int32, (_ROWS, 128), 1))
        key_ref[...] = jnp.where(idx < n_valid, key, -1)

    return pl.pallas_call(
        body,
        out_shape=jax.ShapeDtypeStruct((_ROWS, 128), jnp.int32),
    )(px2, py2)


@functools.cache
def _make_count_kernel():
    mesh = plsc.VectorSubcoreMesh(
        core_axis_name="c", subcore_axis_name="s",
        num_cores=_NC, num_subcores=_NS)

    @functools.partial(
        pl.kernel,
        mesh=mesh,
        out_type=jax.ShapeDtypeStruct((_NW, 16), jnp.float32),
        compiler_params=pltpu.CompilerParams(needs_layout_passes=False),
        scratch_types=[
            pltpu.VMEM((2, _CHUNK), jnp.int32),
            pltpu.VMEM((_BAND_ROWS * _GRID + 16,), jnp.float32),
            pltpu.VMEM((16 * 512,), jnp.float32),
            pltpu.VMEM((8 * 256,), jnp.float32),
            pltpu.VMEM((4 * 128,), jnp.float32),
            pltpu.VMEM((2 * 64,), jnp.float32),
            pltpu.VMEM((1 * 32,), jnp.float32),
            pltpu.VMEM((16,), jnp.float32),
            pltpu.SemaphoreType.DMA,
            pltpu.SemaphoreType.DMA,
        ],
    )
    def count_kernel(keys_hbm, out_hbm, kbuf, occ, p1, p2, p3, p4, p5,
                     res, sem0, sem1):
        wid = lax.axis_index("s") * _NC + lax.axis_index("c")
        wbase = wid * (_BAND_ROWS * _GRID)
        lanes = lax.iota(jnp.int32, 16)
        zero16 = jnp.zeros((16,), jnp.float32)
        ones16 = jnp.ones((16,), jnp.float32)

        @plsc.parallel_loop(0, (_BAND_ROWS * _GRID) // 16, unroll=16)
        def _zero(i):
            occ[pl.ds(i * 16, 16)] = zero16

        sems = (sem0, sem1)
        pltpu.async_copy(keys_hbm.at[0], kbuf.at[0], sem0)

        def chunk_body(h, _):
            for b in range(2):
                c = h * 2 + b
                pltpu.make_async_copy(keys_hbm.at[c], kbuf.at[b],
                                      sems[b]).wait()

                @pl.when(c + 1 < _NCHUNK)
                def _():
                    pltpu.async_copy(keys_hbm.at[c + 1], kbuf.at[1 - b],
                                     sems[1 - b])

                @plsc.parallel_loop(0, _CHUNK // 16, unroll=16)
                def _scan(j):
                    k = kbuf[b, pl.ds(j * 16, 16)]
                    a = k - wbase
                    au = plsc.bitcast(a, jnp.uint32)
                    addr = plsc.bitcast(
                        jnp.minimum(au, jnp.uint32(_BAND_ROWS * _GRID)),
                        jnp.int32)
                    plsc.store_scatter(occ, [addr], ones16)
            return 0
        lax.fori_loop(0, _NCHUNK // 2, chunk_body, 0)

        def pool(src, dst, hd, wd, with_sum):
            gpr = wd // 16
            lg = gpr.bit_length() - 1
            s = 2 * wd
            iota2 = lanes * 2

            @plsc.parallel_loop(0, hd * gpr, unroll=8 if gpr >= 8 else 2,
                                carry=(zero16, zero16))
            def body(cc, carry):
                accm, accs = carry
                yy = lax.shift_right_logical(cc, lg)
                j = lax.bitwise_and(cc, gpr - 1)
                base = yy * (2 * s) + j * 32 + iota2
                a = plsc.load_gather(src, [base])
                b2 = plsc.load_gather(src, [base + 1])
                e = plsc.load_gather(src, [base + s])
                f = plsc.load_gather(src, [base + s + 1])
                m = jnp.maximum(jnp.maximum(a, b2), jnp.maximum(e, f))
                dst[pl.ds(cc * 16, 16)] = m
                accm = accm + m
                if with_sum:
                    accs = accs + ((a + b2) + (e + f))
                return (accm, accs)

            return body

        acc1, acc0 = pool(occ, p1, 16, 512, True)
        acc2, _ = pool(p1, p2, 8, 256, False)
        acc3, _ = pool(p2, p3, 4, 128, False)
        acc4, _ = pool(p3, p4, 2, 64, False)
        acc5, _ = pool(p4, p5, 1, 32, False)

        resv = zero16
        for k_idx, acc in enumerate((acc0, acc1, acc2, acc3, acc4, acc5)):
            t = jnp.sum(acc)
            resv = jnp.where(lanes == k_idx, jnp.broadcast_to(t, (16,)),
                             resv)
        res[...] = resv
        pltpu.sync_copy(res, out_hbm.at[wid])

    return count_kernel


def _combine(t, tt):
    def body(t_ref, tt_ref, pc0_ref, cnt_ref):
        tm = t_ref[...]
        tmt = tt_ref[...]

        tot = jnp.sum(tm, axis=0, keepdims=True)
        pc0_ref[...] = tot[:, 0:3]

        c0 = tmt[0:1] + tmt[1:2] + tmt[2:3] + tmt[3:4]
        av = tm[:, 1:2] + tm[:, 2:3] + tm[:, 3:4] + tm[:, 4:5]
        bv = tm[:, 2:3] + tm[:, 3:4] + tm[:, 4:5] + tm[:, 5:6]
        jj = lax.broadcasted_iota(jnp.int32, (32, 32), 0)
        ss = lax.broadcasted_iota(jnp.int32, (32, 32), 1)
        m1 = ((jj >> 1) == ss).astype(jnp.float32)
        m2 = ((jj >> 2) == ss).astype(jnp.float32)
        c1 = jnp.sum(av * m1, axis=0, keepdims=True)
        c2 = jnp.sum(bv * m2, axis=0, keepdims=True)
        cnt_ref[...] = jnp.concatenate([c0, c1, c2], axis=0)

    return pl.pallas_call(
        body,
        out_shape=[
            jax.ShapeDtypeStruct((1, 3), jnp.float32),
            jax.ShapeDtypeStruct((3, 32), jnp.float32),
        ],
    )(t, tt)


def kernel(points_inds, first_res_idx):
    del first_res_idx
    pts = points_inds
    n = pts.shape[0]
    px = jnp.pad(pts[:, 0], (0, _PADN - n))
    py = jnp.pad(pts[:, 1], (0, _PADN - n))
    keys = _bin_keys(px.reshape(_ROWS, 128), py.reshape(_ROWS, 128), n)
    t = _make_count_kernel()(keys.reshape(_NCHUNK, _CHUNK))
    pc0, counts = _combine(t, t.T)
    return pc0, counts


# speedup vs baseline: 2.3181x; 1.0518x over previous
import functools

import jax
import jax.numpy as jnp
from jax import lax
from jax.experimental import pallas as pl
from jax.experimental.pallas import tpu as pltpu
from jax.experimental.pallas import tpu_sc as plsc

_GRID = 1024
_BAND_ROWS = 32
_PADN = 204800
_ROWS = _PADN // 128
_CHUNK = 25600
_NCHUNK = _PADN // _CHUNK

_NC = 2
_NS = 16
_NW = _NC * _NS


def _bin_keys(px2, py2, n_valid):
    def body(px_ref, py_ref, key_ref):
        x = px_ref[...]
        y = py_ref[...]
        cx = ((x - jnp.float32(-51.2)) / jnp.float32(0.1)).astype(jnp.int32)
        cy = ((y - jnp.float32(-51.2)) / jnp.float32(0.1)).astype(jnp.int32)
        key = (cy << 10) | cx
        idx = (lax.broadcasted_iota(jnp.int32, (_ROWS, 128), 0) * 128
               + lax.broadcasted_iota(jnp.int32, (_ROWS, 128), 1))
        key_ref[...] = jnp.where(idx < n_valid, key, -1)

    return pl.pallas_call(
        body,
        out_shape=jax.ShapeDtypeStruct((_ROWS, 128), jnp.int32),
    )(px2, py2)


@functools.cache
def _make_count_kernel():
    mesh = plsc.VectorSubcoreMesh(
        core_axis_name="c", subcore_axis_name="s",
        num_cores=_NC, num_subcores=_NS)

    @functools.partial(
        pl.kernel,
        mesh=mesh,
        out_type=jax.ShapeDtypeStruct((_NW, 16), jnp.float32),
        compiler_params=pltpu.CompilerParams(needs_layout_passes=False),
        scratch_types=[
            pltpu.VMEM((2, _CHUNK), jnp.int32),
            pltpu.VMEM((_BAND_ROWS * _GRID + 16,), jnp.float32),
            pltpu.VMEM((16 * 512,), jnp.float32),
            pltpu.VMEM((8 * 256,), jnp.float32),
            pltpu.VMEM((4 * 128,), jnp.float32),
            pltpu.VMEM((2 * 64,), jnp.float32),
            pltpu.VMEM((1 * 32,), jnp.float32),
            pltpu.VMEM((16,), jnp.float32),
            pltpu.SemaphoreType.DMA,
            pltpu.SemaphoreType.DMA,
        ],
    )
    def count_kernel(keys_hbm, out_hbm, kbuf, occ, p1, p2, p3, p4, p5,
                     res, sem0, sem1):
        wid = lax.axis_index("s") * _NC + lax.axis_index("c")
        wbase = wid * (_BAND_ROWS * _GRID)
        lanes = lax.iota(jnp.int32, 16)
        zero16 = jnp.zeros((16,), jnp.float32)
        ones16 = jnp.ones((16,), jnp.float32)

        @plsc.parallel_loop(0, (_BAND_ROWS * _GRID) // 16, unroll=16)
        def _zero(i):
            occ[pl.ds(i * 16, 16)] = zero16

        sems = (sem0, sem1)
        pltpu.async_copy(keys_hbm.at[0], kbuf.at[0], sem0)

        def chunk_body(h, _):
            for b in range(2):
                c = h * 2 + b
                pltpu.make_async_copy(keys_hbm.at[c], kbuf.at[b],
                                      sems[b]).wait()

                @pl.when(c + 1 < _NCHUNK)
                def _():
                    pltpu.async_copy(keys_hbm.at[c + 1], kbuf.at[1 - b],
                                     sems[1 - b])

                @plsc.parallel_loop(0, _CHUNK // 16, unroll=16)
                def _scan(j):
                    k = kbuf[b, pl.ds(j * 16, 16)]
                    a = k - wbase
                    au = plsc.bitcast(a, jnp.uint32)
                    addr = plsc.bitcast(
                        jnp.minimum(au, jnp.uint32(_BAND_ROWS * _GRID)),
                        jnp.int32)
                    plsc.store_scatter(occ, [addr], ones16)
            return 0
        lax.fori_loop(0, _NCHUNK // 2, chunk_body, 0)

        def pool(src, dst, hd, wd, with_sum):
            gpr = wd // 16
            lg = gpr.bit_length() - 1
            s = 2 * wd
            iota2 = lanes * 2

            @plsc.parallel_loop(0, hd * gpr, unroll=8 if gpr >= 8 else 2,
                                carry=(zero16, zero16))
            def body(cc, carry):
                accm, accs = carry
                yy = lax.shift_right_logical(cc, lg)
                j = lax.bitwise_and(cc, gpr - 1)
                base = yy * (2 * s) + j * 32 + iota2
                a = plsc.load_gather(src, [base])
                b2 = plsc.load_gather(src, [base + 1])
                e = plsc.load_gather(src, [base + s])
                f = plsc.load_gather(src, [base + s + 1])
                m = jnp.maximum(jnp.maximum(a, b2), jnp.maximum(e, f))
                dst[pl.ds(cc * 16, 16)] = m
                accm = accm + m
                if with_sum:
                    accs = accs + ((a + b2) + (e + f))
                return (accm, accs)

            return body

        acc1, acc0 = pool(occ, p1, 16, 512, True)
        acc2, _ = pool(p1, p2, 8, 256, False)
        acc3, _ = pool(p2, p3, 4, 128, False)
        acc4, _ = pool(p3, p4, 2, 64, False)
        acc5, _ = pool(p4, p5, 1, 32, False)

        resv = zero16
        for k_idx, acc in enumerate((acc0, acc1, acc2, acc3, acc4, acc5)):
            t = jnp.sum(acc)
            resv = jnp.where(lanes == k_idx, jnp.broadcast_to(t, (16,)),
                             resv)
        res[...] = resv
        pltpu.sync_copy(res, out_hbm.at[wid])

    return count_kernel


def _combine(t, tt):
    def body(t_ref, tt_ref, pc0_ref, cnt_ref):
        tm = t_ref[...]
        tmt = tt_ref[...]

        tot = jnp.sum(tm, axis=0, keepdims=True)
        pc0_ref[...] = tot[:, 0:3]

        c0 = tmt[0:1] + tmt[1:2] + tmt[2:3] + tmt[3:4]
        av = tm[:, 1:2] + tm[:, 2:3] + tm[:, 3:4] + tm[:, 4:5]
        bv = tm[:, 2:3] + tm[:, 3:4] + tm[:, 4:5] + tm[:, 5:6]
        jj = lax.broadcasted_iota(jnp.int32, (32, 32), 0)
        ss = lax.broadcasted_iota(jnp.int32, (32, 32), 1)
        m1 = ((jj >> 1) == ss).astype(jnp.float32)
        m2 = ((jj >> 2) == ss).astype(jnp.float32)
        c1 = jnp.sum(av * m1, axis=0, keepdims=True)
        c2 = jnp.sum(bv * m2, axis=0, keepdims=True)
        cnt_ref[...] = jnp.concatenate([c0, c1, c2], axis=0)

    return pl.pallas_call(
        body,
        out_shape=[
            jax.ShapeDtypeStruct((1, 3), jnp.float32),
            jax.ShapeDtypeStruct((3, 32), jnp.float32),
        ],
    )(t, tt)


def kernel(points_inds, first_res_idx):
    del first_res_idx
    pts = points_inds
    n = pts.shape[0]
    px = jnp.pad(pts[:, 0], (0, _PADN - n))
    py = jnp.pad(pts[:, 1], (0, _PADN - n))
    keys = _bin_keys(px.reshape(_ROWS, 128), py.reshape(_ROWS, 128), n)
    t = _make_count_kernel()(keys.reshape(_NCHUNK, _CHUNK))
    pc0, counts = _combine(t, t.T)
    return pc0, counts


# speedup vs baseline: 2.9087x; 1.2548x over previous
import functools

import jax
import jax.numpy as jnp
from jax import lax
from jax.experimental import pallas as pl
from jax.experimental.pallas import tpu as pltpu
from jax.experimental.pallas import tpu_sc as plsc

_GRID = 1024
_BAND_ROWS = 32
_PADN = 204800
_ROWS = _PADN // 128
_CHUNK = 25600
_NCHUNK = _PADN // _CHUNK

_NC = 2
_NS = 16
_NW = _NC * _NS


def _bin_keys(px2, py2, n_valid):
    def body(px_ref, py_ref, key_ref):
        x = px_ref[...]
        y = py_ref[...]
        cx = ((x - jnp.float32(-51.2)) / jnp.float32(0.1)).astype(jnp.int32)
        cy = ((y - jnp.float32(-51.2)) / jnp.float32(0.1)).astype(jnp.int32)
        key = (cy << 10) | cx
        idx = (lax.broadcasted_iota(jnp.int32, (_ROWS, 128), 0) * 128
               + lax.broadcasted_iota(jnp.int32, (_ROWS, 128), 1))
        key_ref[...] = jnp.where(idx < n_valid, key, -1)

    return pl.pallas_call(
        body,
        out_shape=jax.ShapeDtypeStruct((_ROWS, 128), jnp.int32),
    )(px2, py2)


@functools.cache
def _make_count_kernel():
    mesh = plsc.VectorSubcoreMesh(
        core_axis_name="c", subcore_axis_name="s",
        num_cores=_NC, num_subcores=_NS)

    @functools.partial(
        pl.kernel,
        mesh=mesh,
        out_type=jax.ShapeDtypeStruct((_NW, 16), jnp.float32),
        compiler_params=pltpu.CompilerParams(needs_layout_passes=False),
        scratch_types=[
            pltpu.VMEM_SHARED((_PADN,), jnp.int32),
            pltpu.VMEM((2, _CHUNK), jnp.int32),
            pltpu.VMEM((_BAND_ROWS * _GRID + 16,), jnp.float32),
            pltpu.VMEM((16 * 512,), jnp.float32),
            pltpu.VMEM((8 * 256,), jnp.float32),
            pltpu.VMEM((4 * 128,), jnp.float32),
            pltpu.VMEM((2 * 64,), jnp.float32),
            pltpu.VMEM((1 * 32,), jnp.float32),
            pltpu.VMEM((16,), jnp.float32),
            pltpu.SemaphoreType.DMA,
            pltpu.SemaphoreType.DMA,
            pltpu.SemaphoreType.DMA,
        ],
    )
    def count_kernel(keys_hbm, out_hbm, skeys, kbuf, occ, p1, p2, p3, p4,
                     p5, res, sem0, sem1, sem2):
        sid = lax.axis_index("s")
        wid = sid * _NC + lax.axis_index("c")
        wbase = wid * (_BAND_ROWS * _GRID)
        lanes = lax.iota(jnp.int32, 16)
        zero16 = jnp.zeros((16,), jnp.float32)
        ones16 = jnp.ones((16,), jnp.float32)

        nstage = _PADN // _NS
        pltpu.async_copy(keys_hbm.at[pl.ds(sid * nstage, nstage)],
                         skeys.at[pl.ds(sid * nstage, nstage)], sem2)

        @plsc.parallel_loop(0, (_BAND_ROWS * _GRID) // 16, unroll=16)
        def _zero(i):
            occ[pl.ds(i * 16, 16)] = zero16

        pltpu.make_async_copy(keys_hbm.at[pl.ds(sid * nstage, nstage)],
                              skeys.at[pl.ds(sid * nstage, nstage)],
                              sem2).wait()
        plsc.subcore_barrier()

        sems = (sem0, sem1)
        pltpu.async_copy(skeys.at[pl.ds(0, _CHUNK)], kbuf.at[0], sem0)

        def chunk_body(h, _):
            for b in range(2):
                c = h * 2 + b
                pltpu.make_async_copy(skeys.at[pl.ds(c * _CHUNK, _CHUNK)],
                                      kbuf.at[b], sems[b]).wait()

                @pl.when(c + 1 < _NCHUNK)
                def _():
                    pltpu.async_copy(
                        skeys.at[pl.ds((c + 1) * _CHUNK, _CHUNK)],
                        kbuf.at[1 - b], sems[1 - b])

                @plsc.parallel_loop(0, _CHUNK // 16, unroll=16)
                def _scan(j):
                    k = kbuf[b, pl.ds(j * 16, 16)]
                    a = k - wbase
                    au = plsc.bitcast(a, jnp.uint32)
                    addr = plsc.bitcast(
                        jnp.minimum(au, jnp.uint32(_BAND_ROWS * _GRID)),
                        jnp.int32)
                    plsc.store_scatter(occ, [addr], ones16)
            return 0
        lax.fori_loop(0, _NCHUNK // 2, chunk_body, 0)

        def pool(src, dst, hd, wd, with_sum):
            gpr = wd // 16
            lg = gpr.bit_length() - 1
            s = 2 * wd
            iota2 = lanes * 2

            @plsc.parallel_loop(0, hd * gpr, unroll=8 if gpr >= 8 else 2,
                                carry=(zero16, zero16))
            def body(cc, carry):
                accm, accs = carry
                yy = lax.shift_right_logical(cc, lg)
                j = lax.bitwise_and(cc, gpr - 1)
                base = yy * (2 * s) + j * 32 + iota2
                a = plsc.load_gather(src, [base])
                b2 = plsc.load_gather(src, [base + 1])
                e = plsc.load_gather(src, [base + s])
                f = plsc.load_gather(src, [base + s + 1])
                m = jnp.maximum(jnp.maximum(a, b2), jnp.maximum(e, f))
                dst[pl.ds(cc * 16, 16)] = m
                accm = accm + m
                if with_sum:
                    accs = accs + ((a + b2) + (e + f))
                return (accm, accs)

            return body

        acc1, acc0 = pool(occ, p1, 16, 512, True)
        acc2, _ = pool(p1, p2, 8, 256, False)
        acc3, _ = pool(p2, p3, 4, 128, False)
        acc4, _ = pool(p3, p4, 2, 64, False)
        acc5, _ = pool(p4, p5, 1, 32, False)

        resv = zero16
        for k_idx, acc in enumerate((acc0, acc1, acc2, acc3, acc4, acc5)):
            t = jnp.sum(acc)
            resv = jnp.where(lanes == k_idx, jnp.broadcast_to(t, (16,)),
                             resv)
        res[...] = resv
        pltpu.sync_copy(res, out_hbm.at[wid])

    return count_kernel


def _combine(t, tt):
    def body(t_ref, tt_ref, pc0_ref, cnt_ref):
        tm = t_ref[...]
        tmt = tt_ref[...]

        tot = jnp.sum(tm, axis=0, keepdims=True)
        pc0_ref[...] = tot[:, 0:3]

        c0 = tmt[0:1] + tmt[1:2] + tmt[2:3] + tmt[3:4]
        av = tm[:, 1:2] + tm[:, 2:3] + tm[:, 3:4] + tm[:, 4:5]
        bv = tm[:, 2:3] + tm[:, 3:4] + tm[:, 4:5] + tm[:, 5:6]
        jj = lax.broadcasted_iota(jnp.int32, (32, 32), 0)
        ss = lax.broadcasted_iota(jnp.int32, (32, 32), 1)
        m1 = ((jj >> 1) == ss).astype(jnp.float32)
        m2 = ((jj >> 2) == ss).astype(jnp.float32)
        c1 = jnp.sum(av * m1, axis=0, keepdims=True)
        c2 = jnp.sum(bv * m2, axis=0, keepdims=True)
        cnt_ref[...] = jnp.concatenate([c0, c1, c2], axis=0)

    return pl.pallas_call(
        body,
        out_shape=[
            jax.ShapeDtypeStruct((1, 3), jnp.float32),
            jax.ShapeDtypeStruct((3, 32), jnp.float32),
        ],
    )(t, tt)


def kernel(points_inds, first_res_idx):
    del first_res_idx
    pts = points_inds
    n = pts.shape[0]
    px = jnp.pad(pts[:, 0], (0, _PADN - n))
    py = jnp.pad(pts[:, 1], (0, _PADN - n))
    keys = _bin_keys(px.reshape(_ROWS, 128), py.reshape(_ROWS, 128), n)
    t = _make_count_kernel()(keys.reshape(_PADN))
    pc0, counts = _combine(t, t.T)
    return pc0, counts
